# baseline (speedup 1.0000x reference)
import functools

import jax
import jax.numpy as jnp
from jax import lax
from jax.experimental import pallas as pl
from jax.experimental.pallas import tpu as pltpu
from jax.experimental.pallas import tpu_sc as plsc

VOCAB_N = 1000000
HID = 64
BATCH_N = 4096
SEQ_N = 200
GATES = 4 * HID

_NC = 2
_NS = 16
_NW = _NC * _NS

_ROWS = BATCH_N * SEQ_N
_PER_W = _ROWS // _NW
_CH = 1024
_NCHUNK = _PER_W // _CH


def _sc_gather_body(idx_hbm, emb_hbm, out_hbm, idx_v, rows_v, sem):
    wid = lax.axis_index("s") * _NC + lax.axis_index("c")
    base = wid * _PER_W

    def chunk(j, carry):
        off = base + j * _CH
        pltpu.sync_copy(idx_hbm.at[pl.ds(off, _CH)], idx_v)
        pltpu.async_copy(emb_hbm.at[idx_v], rows_v, sem).wait()
        pltpu.sync_copy(rows_v, out_hbm.at[pl.ds(off, _CH)])
        return carry

    lax.fori_loop(0, _NCHUNK, chunk, 0)


@functools.cache
def _sc_gather():
    return functools.partial(
        pl.kernel,
        mesh=plsc.VectorSubcoreMesh(core_axis_name="c", subcore_axis_name="s"),
        compiler_params=pltpu.CompilerParams(use_tc_tiling_on_sc=False),
        out_type=jax.ShapeDtypeStruct((_ROWS, HID), jnp.float32),
        scratch_types=[
            pltpu.VMEM((_CH,), jnp.int32),
            pltpu.VMEM((_CH, HID), jnp.float32),
            pltpu.SemaphoreType.DMA,
        ],
    )(_sc_gather_body)


def _lstm_body(e_ref, w_ref, u_ref, b_ref, h_out, c_out, h_s, c_s):
    t = pl.program_id(0)

    @pl.when(t == 0)
    def _init():
        h_s[...] = jnp.zeros_like(h_s)
        c_s[...] = jnp.zeros_like(c_s)

    xt = e_ref[0]
    h = h_s[...]
    c = c_s[...]
    z = (jnp.dot(xt, w_ref[...], preferred_element_type=jnp.float32)
         + jnp.dot(h, u_ref[...], preferred_element_type=jnp.float32)
         + b_ref[...])
    gi = jax.nn.sigmoid(z[:, :HID])
    gf = jax.nn.sigmoid(z[:, HID:2 * HID])
    gg = jnp.tanh(z[:, 2 * HID:3 * HID])
    go = jax.nn.sigmoid(z[:, 3 * HID:])
    c_new = gf * c + gi * gg
    h_new = go * jnp.tanh(c_new)
    h_s[...] = h_new
    c_s[...] = c_new

    @pl.when(t == SEQ_N - 1)
    def _fin():
        h_out[...] = h_new
        c_out[...] = c_new


_lstm_call = pl.pallas_call(
    _lstm_body,
    grid=(SEQ_N,),
    in_specs=[
        pl.BlockSpec((1, BATCH_N, HID), lambda t: (t, 0, 0)),
        pl.BlockSpec((HID, GATES), lambda t: (0, 0)),
        pl.BlockSpec((HID, GATES), lambda t: (0, 0)),
        pl.BlockSpec((1, GATES), lambda t: (0, 0)),
    ],
    out_specs=[
        pl.BlockSpec((BATCH_N, HID), lambda t: (0, 0)),
        pl.BlockSpec((BATCH_N, HID), lambda t: (0, 0)),
    ],
    out_shape=[
        jax.ShapeDtypeStruct((BATCH_N, HID), jnp.float32),
        jax.ShapeDtypeStruct((BATCH_N, HID), jnp.float32),
    ],
    scratch_shapes=[
        pltpu.VMEM((BATCH_N, HID), jnp.float32),
        pltpu.VMEM((BATCH_N, HID), jnp.float32),
    ],
)


def kernel(x, emb, W, U, b):
    idx = jnp.swapaxes(x, 0, 1).reshape(-1)
    e = _sc_gather()(idx, emb)
    e = e.reshape(SEQ_N, BATCH_N, HID)
    h, c = _lstm_call(e, W, U, b.reshape(1, GATES))
    return (h, h, c)
